# baseline (speedup 1.0000x reference)
import functools

import jax
import jax.numpy as jnp
from jax import lax
from jax.experimental import pallas as pl
from jax.experimental.pallas import tpu as pltpu
from jax.experimental.pallas import tpu_sc as plsc

_NC = 2
_NS = 16
_NW = _NC * _NS

_B = 4 * 8192
_D = 768
_BPW = _B // _NW
_C = 64
_NCHUNK = _BPW // _C


def _emb_body(
    idx_hbm, table_hbm, out_hbm, idx_v, rows0, rows1, sem0, sem1, wsem0, wsem1
):
    wid = lax.axis_index("s") * _NC + lax.axis_index("c")
    base = wid * _BPW
    pltpu.sync_copy(idx_hbm.at[pl.ds(base, _BPW)], idx_v)

    bufs = (rows0, rows1)
    sems = (sem0, sem1)
    wsems = (wsem0, wsem1)
    gathers = [None, None]
    writes = [None, None]
    for g in range(_NCHUNK):
        s = g % 2
        if g >= 2:
            writes[s].wait()
        gathers[s] = pltpu.async_copy(
            table_hbm.at[idx_v.at[pl.ds(g * _C, _C)]], bufs[s], sems[s]
        )
        if g > 0:
            p = (g - 1) % 2
            gathers[p].wait()
            writes[p] = pltpu.async_copy(
                bufs[p], out_hbm.at[pl.ds(base + (g - 1) * _C, _C)], wsems[p]
            )
    last = (_NCHUNK - 1) % 2
    gathers[last].wait()
    writes[last] = pltpu.async_copy(
        bufs[last], out_hbm.at[pl.ds(base + (_NCHUNK - 1) * _C, _C)], wsems[last]
    )
    writes[0].wait()
    writes[1].wait()


_emb_call = pl.kernel(
    _emb_body,
    out_type=jax.ShapeDtypeStruct((_B, _D), jnp.float32),
    mesh=plsc.VectorSubcoreMesh(core_axis_name="c", subcore_axis_name="s"),
    scratch_types=[
        pltpu.VMEM((_BPW,), jnp.int32),
        pltpu.VMEM((_C, _D), jnp.float32),
        pltpu.VMEM((_C, _D), jnp.float32),
        pltpu.SemaphoreType.DMA,
        pltpu.SemaphoreType.DMA,
        pltpu.SemaphoreType.DMA,
        pltpu.SemaphoreType.DMA,
    ],
)


@jax.jit
def kernel(pos_ids, wpe):
    batch, seq = pos_ids.shape
    flat_idx = pos_ids.reshape(-1).astype(jnp.int32)
    out = _emb_call(flat_idx, wpe)
    return out.reshape(batch, seq, wpe.shape[1])


# speedup vs baseline: 1.0057x; 1.0057x over previous
import functools

import jax
import jax.numpy as jnp
from jax import lax
from jax.experimental import pallas as pl
from jax.experimental.pallas import tpu as pltpu
from jax.experimental.pallas import tpu_sc as plsc

_NC = 2
_NS = 16
_NW = _NC * _NS

_B = 4 * 8192
_D = 768
_BPW = _B // _NW
_C = 32
_NCHUNK = _BPW // _C
_NBUF = 4


def _emb_body(idx_hbm, table_hbm, out_hbm, idx_v, bufs, sems, wsems):
    wid = lax.axis_index("s") * _NC + lax.axis_index("c")
    base = wid * _BPW
    pltpu.sync_copy(idx_hbm.at[pl.ds(base, _BPW)], idx_v)

    gathers = [None] * _NBUF
    writes = [None] * _NBUF
    for g in range(_NCHUNK + _NBUF - 1):
        if g < _NCHUNK:
            s = g % _NBUF
            if g >= _NBUF:
                writes[s].wait()
            gathers[s] = pltpu.async_copy(
                table_hbm.at[idx_v.at[pl.ds(g * _C, _C)]], bufs[s], sems[s]
            )
        d = g - (_NBUF - 1)
        if d >= 0:
            p = d % _NBUF
            gathers[p].wait()
            writes[p] = pltpu.async_copy(
                bufs[p], out_hbm.at[pl.ds(base + d * _C, _C)], wsems[p]
            )
    for p in range(_NBUF):
        writes[p].wait()


_emb_call = pl.kernel(
    _emb_body,
    out_type=jax.ShapeDtypeStruct((_B, _D), jnp.float32),
    mesh=plsc.VectorSubcoreMesh(core_axis_name="c", subcore_axis_name="s"),
    scratch_types=[
        pltpu.VMEM((_BPW,), jnp.int32),
        [pltpu.VMEM((_C, _D), jnp.float32) for _ in range(_NBUF)],
        [pltpu.SemaphoreType.DMA for _ in range(_NBUF)],
        [pltpu.SemaphoreType.DMA for _ in range(_NBUF)],
    ],
)


@jax.jit
def kernel(pos_ids, wpe):
    batch, seq = pos_ids.shape
    flat_idx = pos_ids.reshape(-1).astype(jnp.int32)
    out = _emb_call(flat_idx, wpe)
    return out.reshape(batch, seq, wpe.shape[1])
